# baseline (speedup 1.0000x reference)
import jax
import jax.numpy as jnp
from jax import lax
from jax.experimental import pallas as pl
from jax.experimental.pallas import tpu as pltpu
from jax.experimental.pallas import tpu_sc as plsc

N = 10000
D = 128
E = 320000
NC = 2
NS = 16
NW = NC * NS
K = 128
NBUF = 2

FULL = E // NW // K
EPW = FULL * K
REM_BASE = NW * EPW
REM_CHUNKS = (E - REM_BASE) // K
ITERS = FULL // NBUF

STRIP = 624
_PIECES = ((0, 128), (128, 128), (256, 128), (384, 128), (512, 112))
_TAIL_BASE = NS * STRIP
_TAIL = N - _TAIL_BASE


W_SCALE = 2.0 ** -24


def _sc_body(x_hbm, ei_hbm, w_hbm, out0_hbm, out1_hbm, acc,
             colv_all, rowwA, rowwB, wvA, wvB, bufA, bufB,
             gA, gB, sA, sB):
    cid = lax.axis_index("c")
    sid = lax.axis_index("s")
    wid = sid * NC + cid
    nchunks = FULL + jnp.where(wid < REM_CHUNKS, 1, 0)
    bufs = ((bufA, rowwA, wvA, gA, sA), (bufB, rowwB, wvB, gB, sB))

    pltpu.async_copy(
        ei_hbm.at[pl.ds(E + wid * EPW, EPW)], colv_all.at[pl.ds(0, EPW)], gA
    )

    @pl.when(wid < REM_CHUNKS)
    def _():
        rem = E + REM_BASE + wid * K
        pltpu.async_copy(ei_hbm.at[pl.ds(rem, K)], colv_all.at[pl.ds(EPW, K)], gB)

    def src_off(cn):
        return jnp.where(cn < FULL, wid * EPW + cn * K, REM_BASE + wid * K)

    def start_fetch(buf, roww, wv, gsem, cn):
        off = src_off(cn)
        pltpu.async_copy(ei_hbm.at[pl.ds(off, K)], roww, gsem)
        pltpu.async_copy(w_hbm.at[pl.ds(off, K)], wv, gsem)
        pltpu.async_copy(x_hbm.at[colv_all.at[pl.ds(cn * K, K)]], buf, gsem)

    def wait_fetch(buf, roww, wv, gsem):
        pltpu.make_async_copy(ei_hbm.at[pl.ds(0, K)], roww, gsem).wait()
        pltpu.make_async_copy(w_hbm.at[pl.ds(0, K)], wv, gsem).wait()
        pltpu.make_async_copy(x_hbm.at[pl.ds(0, K)], buf, gsem).wait()

    def scale(buf, wv):
        def grp(g, carry):
            wvec = wv[pl.ds(g * 16, 16)]
            for r in range(16):
                wspl = jnp.take_along_axis(
                    wvec, jnp.full((16,), r, jnp.int32), axis=0
                )
                j = g * 16 + r
                for k in range(8):
                    s = pl.ds(k * 16, 16)
                    buf[j, s] = buf[j, s] * wspl
            return carry

        lax.fori_loop(0, K // 16, grp, 0)

    def start_scatter(buf, roww, ssem):
        pltpu.async_copy(buf, acc.at[roww], ssem, add=True)

    def wait_scatter(buf, roww, ssem):
        pltpu.make_async_copy(buf, acc.at[roww], ssem).wait()

    zero = jnp.zeros((16,), jnp.float32)

    def zero_row(i, carry):
        for k in range(8):
            bufA[i, pl.ds(k * 16, 16)] = zero
        return carry

    lax.fori_loop(0, K, zero_row, 0)
    base_row = pl.multiple_of(sid * STRIP, 8)
    for off, sz in _PIECES:
        pltpu.async_copy(
            bufA.at[pl.ds(0, sz)],
            acc.at[pl.ds(pl.multiple_of(base_row + off, 8), sz)],
            sA,
        )

    @pl.when(sid == 0)
    def _():
        pltpu.async_copy(bufA.at[pl.ds(0, _TAIL)], acc.at[pl.ds(_TAIL_BASE, _TAIL)], sA)

    for off, sz in _PIECES:
        pltpu.make_async_copy(
            bufA.at[pl.ds(0, sz)],
            acc.at[pl.ds(pl.multiple_of(base_row + off, 8), sz)],
            sA,
        ).wait()

    @pl.when(sid == 0)
    def _():
        pltpu.make_async_copy(
            bufA.at[pl.ds(0, _TAIL)], acc.at[pl.ds(_TAIL_BASE, _TAIL)], sA
        ).wait()

    pltpu.make_async_copy(
        ei_hbm.at[pl.ds(0, EPW)], colv_all.at[pl.ds(0, EPW)], gA
    ).wait()

    @pl.when(wid < REM_CHUNKS)
    def _():
        pltpu.make_async_copy(
            ei_hbm.at[pl.ds(0, K)], colv_all.at[pl.ds(EPW, K)], gB
        ).wait()

    plsc.subcore_barrier()

    for i, (buf, roww, wv, gsem, _) in enumerate(bufs):
        start_fetch(buf, roww, wv, gsem, jnp.int32(i))

    def pipe_iter(c2, carry):
        c = c2 * NBUF
        for i, (buf, roww, wv, gsem, ssem) in enumerate(bufs):
            wait_fetch(buf, roww, wv, gsem)
            scale(buf, wv)
            start_scatter(buf, roww, ssem)
        for i, (buf, roww, wv, gsem, ssem) in enumerate(bufs):
            wait_scatter(buf, roww, ssem)

            @pl.when(c + NBUF + i < nchunks)
            def _():
                start_fetch(buf, roww, wv, gsem, c + NBUF + i)

        return carry

    lax.fori_loop(0, ITERS, pipe_iter, 0)

    @pl.when(nchunks > FULL)
    def _():
        buf, roww, wv, gsem, ssem = bufs[0]
        wait_fetch(buf, roww, wv, gsem)
        scale(buf, wv)
        start_scatter(buf, roww, ssem)
        wait_scatter(buf, roww, ssem)

    plsc.subcore_barrier()

    def export_piece(out_hbm, issue, off, sz):
        o = pl.multiple_of(off, 8)
        cp = pltpu.make_async_copy(acc.at[pl.ds(o, sz)], out_hbm.at[pl.ds(o, sz)], sA)
        cp.start() if issue else cp.wait()

    for out_hbm, core in ((out0_hbm, 0), (out1_hbm, 1)):
        @pl.when(cid == core)
        def _():
            for issue in (True, False):
                for off, sz in _PIECES:
                    export_piece(out_hbm, issue, base_row + off, sz)

                @pl.when(sid == 0)
                def _():
                    export_piece(out_hbm, issue, _TAIL_BASE, _TAIL)


BN = 2000


def _norm_body(p0_ref, p1_ref, o_ref):
    s = p0_ref[...] + p1_ref[...]
    inner = jnp.sum(s * s, axis=1, keepdims=True) - 2.0 * (s[:, 0:1] ** 2)
    coeff = lax.rsqrt(jnp.abs(inner))
    o_ref[...] = s * coeff


def _normalize(p0, p1):
    return pl.pallas_call(
        _norm_body,
        grid=(N // BN,),
        in_specs=[
            pl.BlockSpec((BN, D), lambda i: (i, 0)),
            pl.BlockSpec((BN, D), lambda i: (i, 0)),
        ],
        out_specs=pl.BlockSpec((BN, D), lambda i: (i, 0)),
        out_shape=jax.ShapeDtypeStruct((N, D), jnp.float32),
    )(p0, p1)


def kernel(x, edge_index, edge_values):
    ei_flat = edge_index.reshape(2 * E)
    mesh = plsc.VectorSubcoreMesh(core_axis_name="c", subcore_axis_name="s")
    p0, p1 = pl.kernel(
        _sc_body,
        out_type=[
            jax.ShapeDtypeStruct((N, D), jnp.float32),
            jax.ShapeDtypeStruct((N, D), jnp.float32),
        ],
        mesh=mesh,
        scratch_types=[
            pltpu.VMEM_SHARED((N, D), jnp.float32),
            pltpu.VMEM((EPW + K,), jnp.int32),
            pltpu.VMEM((K,), jnp.int32),
            pltpu.VMEM((K,), jnp.int32),
            pltpu.VMEM((K,), jnp.float32),
            pltpu.VMEM((K,), jnp.float32),
            pltpu.VMEM((K, D), jnp.float32),
            pltpu.VMEM((K, D), jnp.float32),
            pltpu.SemaphoreType.DMA,
            pltpu.SemaphoreType.DMA,
            pltpu.SemaphoreType.DMA,
            pltpu.SemaphoreType.DMA,
        ],
    )(x, ei_flat, edge_values)
    return _normalize(p0, p1)


# speedup vs baseline: 1.0650x; 1.0650x over previous
import jax
import jax.numpy as jnp
from jax import lax
from jax.experimental import pallas as pl
from jax.experimental.pallas import tpu as pltpu
from jax.experimental.pallas import tpu_sc as plsc

N = 10000
D = 128
E = 320000
NC = 2
NS = 16
NW = NC * NS
K = 128
NBUF = 2

FULL = E // NW // K
EPW = FULL * K
REM_BASE = NW * EPW
REM_CHUNKS = (E - REM_BASE) // K
ITERS = FULL // NBUF

STRIP = 624
_PIECES = ((0, 128), (128, 128), (256, 128), (384, 128), (512, 112))
_TAIL_BASE = NS * STRIP
_TAIL = N - _TAIL_BASE


W_SCALE = 2.0 ** -24


def _sc_body(x_hbm, ei_hbm, w_hbm, out0_hbm, out1_hbm, acc,
             colv_all, rowwA, rowwB, wvA, wvB, bufA, bufB,
             gA, gB, sA, sB):
    cid = lax.axis_index("c")
    sid = lax.axis_index("s")
    wid = sid * NC + cid
    nchunks = FULL + jnp.where(wid < REM_CHUNKS, 1, 0)
    bufs = ((bufA, rowwA, wvA, gA, sA), (bufB, rowwB, wvB, gB, sB))

    pltpu.async_copy(
        ei_hbm.at[pl.ds(E + wid * EPW, EPW)], colv_all.at[pl.ds(0, EPW)], gA
    )

    @pl.when(wid < REM_CHUNKS)
    def _():
        rem = E + REM_BASE + wid * K
        pltpu.async_copy(ei_hbm.at[pl.ds(rem, K)], colv_all.at[pl.ds(EPW, K)], gB)

    def src_off(cn):
        return jnp.where(cn < FULL, wid * EPW + cn * K, REM_BASE + wid * K)

    def start_fetch(buf, roww, wv, gsem, cn):
        off = src_off(cn)
        pltpu.async_copy(ei_hbm.at[pl.ds(off, K)], roww, gsem)
        pltpu.async_copy(w_hbm.at[pl.ds(off, K)], wv, gsem)
        pltpu.async_copy(x_hbm.at[colv_all.at[pl.ds(cn * K, K)]], buf, gsem)

    def wait_fetch(buf, roww, wv, gsem):
        pltpu.make_async_copy(ei_hbm.at[pl.ds(0, K)], roww, gsem).wait()
        pltpu.make_async_copy(w_hbm.at[pl.ds(0, K)], wv, gsem).wait()
        pltpu.make_async_copy(x_hbm.at[pl.ds(0, K)], buf, gsem).wait()

    def scale(buf, wv):
        def grp(g, carry):
            wvec = wv[pl.ds(g * 16, 16)]
            for r in range(16):
                wspl = jnp.take_along_axis(
                    wvec, jnp.full((16,), r, jnp.int32), axis=0
                )
                j = g * 16 + r
                for k in range(8):
                    s = pl.ds(k * 16, 16)
                    buf[j, s] = buf[j, s] * wspl
            return carry

        lax.fori_loop(0, K // 16, grp, 0)

    def start_scatter(buf, roww, ssem):
        pltpu.async_copy(buf, acc.at[roww], ssem, add=True)

    def wait_scatter(buf, roww, ssem):
        pltpu.make_async_copy(buf, acc.at[roww], ssem).wait()

    zero = jnp.zeros((16,), jnp.float32)

    def zero_row(i, carry):
        for k in range(8):
            bufA[i, pl.ds(k * 16, 16)] = zero
        return carry

    lax.fori_loop(0, K, zero_row, 0)
    base_row = pl.multiple_of(sid * STRIP, 8)
    for off, sz in _PIECES:
        pltpu.async_copy(
            bufA.at[pl.ds(0, sz)],
            acc.at[pl.ds(pl.multiple_of(base_row + off, 8), sz)],
            sA,
        )

    @pl.when(sid == 0)
    def _():
        pltpu.async_copy(bufA.at[pl.ds(0, _TAIL)], acc.at[pl.ds(_TAIL_BASE, _TAIL)], sA)

    pltpu.make_async_copy(
        ei_hbm.at[pl.ds(0, EPW)], colv_all.at[pl.ds(0, EPW)], gA
    ).wait()

    @pl.when(wid < REM_CHUNKS)
    def _():
        pltpu.make_async_copy(
            ei_hbm.at[pl.ds(0, K)], colv_all.at[pl.ds(EPW, K)], gB
        ).wait()

    start_fetch(bufB, rowwB, wvB, gB, jnp.int32(1))

    for off, sz in _PIECES:
        pltpu.make_async_copy(
            bufA.at[pl.ds(0, sz)],
            acc.at[pl.ds(pl.multiple_of(base_row + off, 8), sz)],
            sA,
        ).wait()

    @pl.when(sid == 0)
    def _():
        pltpu.make_async_copy(
            bufA.at[pl.ds(0, _TAIL)], acc.at[pl.ds(_TAIL_BASE, _TAIL)], sA
        ).wait()

    start_fetch(bufA, rowwA, wvA, gA, jnp.int32(0))

    plsc.subcore_barrier()


    def pipe_iter(c2, carry):
        c = c2 * NBUF
        for i, (buf, roww, wv, gsem, ssem) in enumerate(bufs):
            wait_fetch(buf, roww, wv, gsem)
            scale(buf, wv)
            start_scatter(buf, roww, ssem)
        for i, (buf, roww, wv, gsem, ssem) in enumerate(bufs):
            wait_scatter(buf, roww, ssem)

            @pl.when(c + NBUF + i < nchunks)
            def _():
                start_fetch(buf, roww, wv, gsem, c + NBUF + i)

        return carry

    lax.fori_loop(0, ITERS, pipe_iter, 0)

    @pl.when(nchunks > FULL)
    def _():
        buf, roww, wv, gsem, ssem = bufs[0]
        wait_fetch(buf, roww, wv, gsem)
        scale(buf, wv)
        start_scatter(buf, roww, ssem)
        wait_scatter(buf, roww, ssem)

    plsc.subcore_barrier()

    def export_piece(out_hbm, issue, off, sz):
        o = pl.multiple_of(off, 8)
        cp = pltpu.make_async_copy(acc.at[pl.ds(o, sz)], out_hbm.at[pl.ds(o, sz)], sA)
        cp.start() if issue else cp.wait()

    for out_hbm, core in ((out0_hbm, 0), (out1_hbm, 1)):
        @pl.when(cid == core)
        def _():
            for issue in (True, False):
                for off, sz in _PIECES:
                    export_piece(out_hbm, issue, base_row + off, sz)

                @pl.when(sid == 0)
                def _():
                    export_piece(out_hbm, issue, _TAIL_BASE, _TAIL)


BN = 2000


def _norm_body(p0_ref, p1_ref, o_ref):
    s = p0_ref[...] + p1_ref[...]
    inner = jnp.sum(s * s, axis=1, keepdims=True) - 2.0 * (s[:, 0:1] ** 2)
    coeff = lax.rsqrt(jnp.abs(inner))
    o_ref[...] = s * coeff


def _normalize(p0, p1):
    return pl.pallas_call(
        _norm_body,
        grid=(N // BN,),
        in_specs=[
            pl.BlockSpec((BN, D), lambda i: (i, 0)),
            pl.BlockSpec((BN, D), lambda i: (i, 0)),
        ],
        out_specs=pl.BlockSpec((BN, D), lambda i: (i, 0)),
        out_shape=jax.ShapeDtypeStruct((N, D), jnp.float32),
    )(p0, p1)


def kernel(x, edge_index, edge_values):
    ei_flat = edge_index.reshape(2 * E)
    mesh = plsc.VectorSubcoreMesh(core_axis_name="c", subcore_axis_name="s")
    p0, p1 = pl.kernel(
        _sc_body,
        out_type=[
            jax.ShapeDtypeStruct((N, D), jnp.float32),
            jax.ShapeDtypeStruct((N, D), jnp.float32),
        ],
        mesh=mesh,
        scratch_types=[
            pltpu.VMEM_SHARED((N, D), jnp.float32),
            pltpu.VMEM((EPW + K,), jnp.int32),
            pltpu.VMEM((K,), jnp.int32),
            pltpu.VMEM((K,), jnp.int32),
            pltpu.VMEM((K,), jnp.float32),
            pltpu.VMEM((K,), jnp.float32),
            pltpu.VMEM((K, D), jnp.float32),
            pltpu.VMEM((K, D), jnp.float32),
            pltpu.SemaphoreType.DMA,
            pltpu.SemaphoreType.DMA,
            pltpu.SemaphoreType.DMA,
            pltpu.SemaphoreType.DMA,
        ],
    )(x, ei_flat, edge_values)
    return _normalize(p0, p1)
